# baseline (speedup 1.0000x reference)
import functools

import jax
import jax.numpy as jnp
from jax import lax
from jax.experimental import pallas as pl
from jax.experimental.pallas import tpu as pltpu
from jax.experimental.pallas import tpu_sc as plsc

_D = 64
_R = 8
_SCALE = 2.0
_LANES = 16
_NDC = _D // _LANES
_BSZ = 256


@functools.cache
def _make_sc_kernel(n_b: int, n_l: int):
    info = plsc.get_sparse_core_info()
    nc, ns = info.num_cores, info.num_subcores
    nw = nc * ns
    bpw = n_b // nw
    ncb = bpw // _BSZ
    assert bpw * nw == n_b and ncb * _BSZ == bpw
    n_chunks = n_l * ncb
    mesh = plsc.VectorSubcoreMesh(core_axis_name="c", subcore_axis_name="s")

    rows_t = pltpu.VMEM((_BSZ, _D), jnp.float32)
    arows_t = pltpu.VMEM((_BSZ, _R), jnp.int32)
    tbuf_t = pltpu.VMEM((_D, _BSZ), jnp.float32)

    @functools.partial(
        pl.kernel,
        mesh=mesh,
        compiler_params=pltpu.CompilerParams(use_tc_tiling_on_sc=False,
                                             needs_layout_passes=False),
        out_type=jax.ShapeDtypeStruct((n_l, _D, n_b), jnp.float32),
        scratch_types=(
            [pltpu.VMEM((n_l, bpw), jnp.int32)]
            + [rows_t] * 2 + [arows_t] * 2 + [tbuf_t] * 2
            + [pltpu.VMEM((_R, _D), jnp.float32)]
            + [pltpu.SemaphoreType.DMA] * 6
        ),
    )
    def k(idxT_hbm, w_hbm, a_hbm, b_hbm, out_hbm, idx_v,
          rows0, rows1, arows0, arows1, tbuf0, tbuf1, b_v,
          semw0, semw1, sema0, sema1, semo0, semo1):
        rows = (rows0, rows1)
        arows = (arows0, arows1)
        tbuf = (tbuf0, tbuf1)
        semw = (semw0, semw1)
        sema = (sema0, sema1)
        semo = (semo0, semo1)

        wid = lax.axis_index("s") * nc + lax.axis_index("c")
        bw0 = wid * bpw
        pltpu.sync_copy(idxT_hbm.at[:, pl.ds(bw0, bpw)], idx_v)
        pltpu.sync_copy(b_hbm, b_v)
        b_vecs = [[plsc.pack(b_v[r, pl.ds(h * 32, _LANES)] * _SCALE,
                             b_v[r, pl.ds(h * 32 + _LANES, _LANES)] * _SCALE,
                             format=plsc.PackFormat.INTERLEAVED)
                   for h in range(2)] for r in range(_R)]
        r_ids = [jnp.full((_LANES,), r, jnp.int32) for r in range(_R)]
        lane = lax.iota(jnp.int32, _LANES)
        d_ids = [lane + c * _LANES for c in range(_NDC)]

        def issue(l, cb):
            idx_slice = idx_v.at[l, pl.ds(cb * _BSZ, _BSZ)]
            pltpu.async_copy(w_hbm.at[idx_slice], rows[cb], semw[cb])
            pltpu.async_copy(a_hbm.at[idx_slice], arows[cb], sema[cb])

        def wait(l, cb):
            idx_slice = idx_v.at[l, pl.ds(cb * _BSZ, _BSZ)]
            pltpu.make_async_copy(w_hbm.at[idx_slice], rows[cb],
                                  semw[cb]).wait()
            pltpu.make_async_copy(a_hbm.at[idx_slice], arows[cb],
                                  sema[cb]).wait()

        def out_slice(l, cb):
            return out_hbm.at[l, :, pl.ds(bw0 + cb * _BSZ, _BSZ)]

        def compute(cb):
            rows_v, arows_v, tb = rows[cb], arows[cb], tbuf[cb]

            @plsc.parallel_loop(0, _BSZ, unroll=4)
            def row_body(i):
                ib = jnp.broadcast_to(i, (_LANES,))
                splats = [
                    jnp.reshape(
                        plsc.bitcast(plsc.load_gather(arows_v,
                                                      [ib, r_ids[r]]),
                                     jnp.bfloat16), (32,))
                    for r in range(_R)
                ]
                for h in range(2):
                    acc = splats[0] * b_vecs[0][h]
                    for r in range(1, _R):
                        acc = acc + splats[r] * b_vecs[r][h]
                    lo, hi = plsc.unpack(acc,
                                         format=plsc.PackFormat.INTERLEAVED)
                    for c, corr in ((2 * h, lo), (2 * h + 1, hi)):
                        fused = rows_v[i, pl.ds(c * _LANES, _LANES)] + corr
                        plsc.store_scatter(tb, [d_ids[c], ib], fused)

        issue(0, 0)

        def l_body(l, carry):
            for cb in range(ncb):
                nxt_cb = (cb + 1) % ncb
                if cb + 1 < ncb:
                    issue(l, nxt_cb)
                else:
                    @pl.when(l + 1 < n_l)
                    def _():
                        issue(l + 1, nxt_cb)

                wait(l, cb)

                @pl.when(l > 0)
                def _():
                    pltpu.make_async_copy(tbuf[cb], out_slice(l - 1, cb),
                                          semo[cb]).wait()

                compute(cb)
                pltpu.async_copy(tbuf[cb], out_slice(l, cb), semo[cb])
            return carry

        lax.fori_loop(0, n_l, l_body, 0)
        for cb in range(ncb):
            pltpu.make_async_copy(tbuf[cb], out_slice(n_l - 1, cb),
                                  semo[cb]).wait()

    return k


def kernel(input, weight, lora_A, lora_B):
    n_b, n_l = input.shape
    idx_t = input.T.astype(jnp.int32)
    a_bf = lora_A.astype(jnp.bfloat16)
    a_dup = jax.lax.bitcast_convert_type(
        jnp.stack([a_bf, a_bf], axis=-1), jnp.int32)
    out = _make_sc_kernel(n_b, n_l)(idx_t, weight, a_dup, lora_B)
    return jnp.transpose(out, (2, 0, 1))


# speedup vs baseline: 1.3970x; 1.3970x over previous
import functools

import jax
import jax.numpy as jnp
from jax import lax
from jax.experimental import pallas as pl
from jax.experimental.pallas import tpu as pltpu
from jax.experimental.pallas import tpu_sc as plsc

_D = 64
_R = 8
_SCALE = 2.0
_LANES = 16
_NDC = _D // _LANES
_BSZ = 256


@functools.cache
def _make_sc_kernel(n_b: int, n_l: int):
    info = plsc.get_sparse_core_info()
    nc, ns = info.num_cores, info.num_subcores
    nw = nc * ns
    bpw = n_b // nw
    ncb = bpw // _BSZ
    assert bpw * nw == n_b and ncb * _BSZ == bpw
    n_chunks = n_l * ncb
    mesh = plsc.VectorSubcoreMesh(core_axis_name="c", subcore_axis_name="s")

    rows_t = pltpu.VMEM((_BSZ, _D), jnp.float32)
    arows_t = pltpu.VMEM((_BSZ, _R), jnp.int32)
    tbuf_t = pltpu.VMEM((_D, _BSZ + 1), jnp.float32)

    @functools.partial(
        pl.kernel,
        mesh=mesh,
        compiler_params=pltpu.CompilerParams(use_tc_tiling_on_sc=False,
                                             needs_layout_passes=False),
        out_type=jax.ShapeDtypeStruct((n_l, _D, n_b), jnp.float32),
        scratch_types=(
            [pltpu.VMEM((n_l, bpw), jnp.int32)]
            + [rows_t] * 2 + [arows_t] * 2 + [tbuf_t] * 2
            + [pltpu.VMEM((_R, _D), jnp.float32)]
            + [pltpu.SemaphoreType.DMA] * 6
        ),
    )
    def k(idxT_hbm, w_hbm, a_hbm, b_hbm, out_hbm, idx_v,
          rows0, rows1, arows0, arows1, tbuf0, tbuf1, b_v,
          semw0, semw1, sema0, sema1, semo0, semo1):
        rows = (rows0, rows1)
        arows = (arows0, arows1)
        tbuf = (tbuf0, tbuf1)
        semw = (semw0, semw1)
        sema = (sema0, sema1)
        semo = (semo0, semo1)

        wid = lax.axis_index("s") * nc + lax.axis_index("c")
        bw0 = wid * bpw
        pltpu.sync_copy(idxT_hbm.at[:, pl.ds(bw0, bpw)], idx_v)
        pltpu.sync_copy(b_hbm, b_v)
        b_vecs = [[plsc.pack(b_v[r, pl.ds(h * 32, _LANES)] * _SCALE,
                             b_v[r, pl.ds(h * 32 + _LANES, _LANES)] * _SCALE,
                             format=plsc.PackFormat.INTERLEAVED)
                   for h in range(2)] for r in range(_R)]
        r_ids = [jnp.full((_LANES,), r, jnp.int32) for r in range(_R)]
        lane = lax.iota(jnp.int32, _LANES)
        d_ids = [lane + c * _LANES for c in range(_NDC)]

        def issue(l, cb):
            idx_slice = idx_v.at[l, pl.ds(cb * _BSZ, _BSZ)]
            pltpu.async_copy(w_hbm.at[idx_slice], rows[cb], semw[cb])
            pltpu.async_copy(a_hbm.at[idx_slice], arows[cb], sema[cb])

        def wait(l, cb):
            idx_slice = idx_v.at[l, pl.ds(cb * _BSZ, _BSZ)]
            pltpu.make_async_copy(w_hbm.at[idx_slice], rows[cb],
                                  semw[cb]).wait()
            pltpu.make_async_copy(a_hbm.at[idx_slice], arows[cb],
                                  sema[cb]).wait()

        def out_slice(l, cb):
            return out_hbm.at[l, :, pl.ds(bw0 + cb * _BSZ, _BSZ)]

        def compute(cb):
            rows_v, arows_v, tb = rows[cb], arows[cb], tbuf[cb]

            @plsc.parallel_loop(0, _BSZ, unroll=4)
            def row_body(i):
                ib = jnp.broadcast_to(i, (_LANES,))
                splats = [
                    jnp.reshape(
                        plsc.bitcast(plsc.load_gather(arows_v,
                                                      [ib, r_ids[r]]),
                                     jnp.bfloat16), (32,))
                    for r in range(_R)
                ]
                for h in range(2):
                    acc = splats[0] * b_vecs[0][h]
                    for r in range(1, _R):
                        acc = acc + splats[r] * b_vecs[r][h]
                    lo, hi = plsc.unpack(acc,
                                         format=plsc.PackFormat.INTERLEAVED)
                    for c, corr in ((2 * h, lo), (2 * h + 1, hi)):
                        fused = rows_v[i, pl.ds(c * _LANES, _LANES)] + corr
                        plsc.store_scatter(tb, [d_ids[c], ib], fused)

        issue(0, 0)

        def l_body(l, carry):
            for cb in range(ncb):
                nxt_cb = (cb + 1) % ncb
                if cb + 1 < ncb:
                    issue(l, nxt_cb)
                else:
                    @pl.when(l + 1 < n_l)
                    def _():
                        issue(l + 1, nxt_cb)

                wait(l, cb)

                @pl.when(l > 0)
                def _():
                    pltpu.make_async_copy(tbuf[cb].at[:, pl.ds(0, _BSZ)],
                                          out_slice(l - 1, cb),
                                          semo[cb]).wait()

                compute(cb)
                pltpu.async_copy(tbuf[cb].at[:, pl.ds(0, _BSZ)],
                                 out_slice(l, cb), semo[cb])
            return carry

        lax.fori_loop(0, n_l, l_body, 0)
        for cb in range(ncb):
            pltpu.make_async_copy(tbuf[cb].at[:, pl.ds(0, _BSZ)],
                                  out_slice(n_l - 1, cb),
                                  semo[cb]).wait()

    return k


def kernel(input, weight, lora_A, lora_B):
    n_b, n_l = input.shape
    idx_t = input.T.astype(jnp.int32)
    a_bf = lora_A.astype(jnp.bfloat16)
    a_dup = jax.lax.bitcast_convert_type(
        jnp.stack([a_bf, a_bf], axis=-1), jnp.int32)
    out = _make_sc_kernel(n_b, n_l)(idx_t, weight, a_dup, lora_B)
    return jnp.transpose(out, (2, 0, 1))
